# Initial kernel scaffold; baseline (speedup 1.0000x reference)
import functools

import jax
import jax.numpy as jnp
from jax import lax
from jax.experimental import pallas as pl
from jax.experimental.pallas import tpu as pltpu
from jax.experimental.pallas import tpu_sc as plsc

N = 10000
E = 320000
IN_CH = 128
HID_CH = 128
OUT_CH = 64
K_PROP = 10
ALPHA = 0.1

NC = 2
NS = 16
NW = NC * NS
NP = 10240
RPT = NP // NS
EPT = E // NW
CHUNK = 80
NCHUNK = EPT // CHUNK

_MESH = plsc.VectorSubcoreMesh(core_axis_name="c", subcore_axis_name="s")


def _zero_vmem(ref, rows, width):
    z = jnp.zeros((16,), jnp.float32)

    def body(i, _):
        for k in range(width // 16):
            ref[i, pl.ds(k * 16, 16)] = z
        return 0

    lax.fori_loop(0, rows, body, 0)


def _deg_body(dst_hbm, pdeg_hbm, idx_v, ones_v, row_v, deg_sh):
    cid = lax.axis_index("c")
    sid = lax.axis_index("s")
    wid = cid * NS + sid

    pltpu.sync_copy(dst_hbm.at[wid], idx_v)
    one = jnp.ones((16,), jnp.float32)
    z = jnp.zeros((16,), jnp.float32)
    for k in range(CHUNK // 16):
        ones_v[pl.ds(k * 16, 16)] = one

    def zrow(i, _):
        row_v[pl.ds(i * 16, 16)] = z
        return 0

    lax.fori_loop(0, RPT // 16, zrow, 0)
    pltpu.sync_copy(row_v, deg_sh.at[pl.ds(sid * RPT, RPT)])
    plsc.subcore_barrier()

    def chunk(j, _):
        pltpu.sync_copy(ones_v, deg_sh.at[idx_v.at[j]], add=True)
        return 0

    lax.fori_loop(0, NCHUNK, chunk, 0)
    plsc.subcore_barrier()
    pltpu.sync_copy(deg_sh.at[pl.ds(sid * RPT, RPT)], row_v)
    pltpu.sync_copy(row_v, pdeg_hbm.at[cid, pl.ds(sid * RPT, RPT)])


_deg_kernel = functools.partial(
    pl.kernel,
    out_type=jax.ShapeDtypeStruct((NC, NP), jnp.float32),
    mesh=_MESH,
    scratch_types=[
        pltpu.VMEM((NCHUNK, CHUNK), jnp.int32),
        pltpu.VMEM((CHUNK,), jnp.float32),
        pltpu.VMEM((RPT,), jnp.float32),
        pltpu.VMEM_SHARED((NP,), jnp.float32),
    ],
)(_deg_body)


def _prop_body(v_hbm, src_hbm, dst_hbm, ps_hbm, idx_s, idx_d, gbuf0, gbuf1,
               zbuf, s_sh, sem0, sem1):
    cid = lax.axis_index("c")
    sid = lax.axis_index("s")
    wid = cid * NS + sid

    pltpu.sync_copy(src_hbm.at[wid], idx_s)
    pltpu.sync_copy(dst_hbm.at[wid], idx_d)

    _zero_vmem(zbuf, RPT // 4, OUT_CH)
    for q in range(4):
        pltpu.sync_copy(zbuf, s_sh.at[pl.ds(sid * RPT + q * (RPT // 4),
                                            RPT // 4)])
    plsc.subcore_barrier()

    pltpu.async_copy(v_hbm.at[idx_s.at[0]], gbuf0, sem0).wait()
    pltpu.async_copy(v_hbm.at[idx_s.at[1]], gbuf1, sem1)

    def chunk(i, _):
        j = 2 * i
        pltpu.sync_copy(gbuf0, s_sh.at[idx_d.at[j]], add=True)
        pltpu.make_async_copy(v_hbm.at[idx_s.at[j + 1]], gbuf1, sem1).wait()
        cp0 = pltpu.async_copy(v_hbm.at[idx_s.at[j + 2]], gbuf0, sem0)
        pltpu.sync_copy(gbuf1, s_sh.at[idx_d.at[j + 1]], add=True)
        cp0.wait()
        pltpu.async_copy(v_hbm.at[idx_s.at[j + 3]], gbuf1, sem1)
        return 0

    lax.fori_loop(0, (NCHUNK - 3) // 2, chunk, 0)
    pltpu.sync_copy(gbuf0, s_sh.at[idx_d.at[NCHUNK - 3]], add=True)
    cp = pltpu.async_copy(v_hbm.at[idx_s.at[NCHUNK - 1]], gbuf0, sem0)
    pltpu.make_async_copy(v_hbm.at[idx_s.at[NCHUNK - 2]], gbuf1, sem1).wait()
    pltpu.sync_copy(gbuf1, s_sh.at[idx_d.at[NCHUNK - 2]], add=True)
    cp.wait()
    pltpu.sync_copy(gbuf0, s_sh.at[idx_d.at[NCHUNK - 1]], add=True)

    plsc.subcore_barrier()
    for q in range(4):
        r0 = sid * RPT + q * (RPT // 4)
        pltpu.sync_copy(s_sh.at[pl.ds(r0, RPT // 4)], zbuf)
        pltpu.sync_copy(zbuf, ps_hbm.at[cid, pl.ds(r0, RPT // 4)])


_prop_kernel = functools.partial(
    pl.kernel,
    out_type=jax.ShapeDtypeStruct((NC, NP, OUT_CH), jnp.float32),
    mesh=_MESH,
    scratch_types=[
        pltpu.VMEM((NCHUNK, CHUNK), jnp.int32),
        pltpu.VMEM((NCHUNK, CHUNK), jnp.int32),
        pltpu.VMEM((CHUNK, OUT_CH), jnp.float32),
        pltpu.VMEM((CHUNK, OUT_CH), jnp.float32),
        pltpu.VMEM((RPT // 4, OUT_CH), jnp.float32),
        pltpu.VMEM_SHARED((NP, OUT_CH), jnp.float32),
        pltpu.SemaphoreType.DMA,
        pltpu.SemaphoreType.DMA,
    ],
)(_prop_body)


_BLK = NP // 16


def _mlp_body(x_r, w1_r, b1_r, w2_r, b2_r, pd0_r, pd1_r, v0_r, d2_r, sq_r):
    h = lax.dot_general(x_r[...], w1_r[...], (((1,), (1,)), ((), ())),
                        preferred_element_type=jnp.float32)
    h = jnp.maximum(h + b1_r[...], 0.0)
    h0 = lax.dot_general(h, w2_r[...], (((1,), (1,)), ((), ())),
                         preferred_element_type=jnp.float32) + b2_r[...]
    deg = pd0_r[...] + pd1_r[...] + 1.0
    dinv = lax.rsqrt(deg)
    d2_r[...] = 0.9 * dinv * dinv
    sq_r[...] = deg * dinv
    v0_r[...] = h0 * dinv


def _mlp_call(xp, W1, b1, W2, b2, pd0, pd1):
    return pl.pallas_call(
        _mlp_body,
        grid=(16,),
        in_specs=[
            pl.BlockSpec((_BLK, IN_CH), lambda i: (i, 0)),
            pl.BlockSpec((HID_CH, IN_CH), lambda i: (0, 0)),
            pl.BlockSpec((1, HID_CH), lambda i: (0, 0)),
            pl.BlockSpec((OUT_CH, HID_CH), lambda i: (0, 0)),
            pl.BlockSpec((1, OUT_CH), lambda i: (0, 0)),
            pl.BlockSpec((_BLK, 1), lambda i: (i, 0)),
            pl.BlockSpec((_BLK, 1), lambda i: (i, 0)),
        ],
        out_specs=[
            pl.BlockSpec((_BLK, OUT_CH), lambda i: (i, 0)),
            pl.BlockSpec((_BLK, 1), lambda i: (i, 0)),
            pl.BlockSpec((_BLK, 1), lambda i: (i, 0)),
        ],
        out_shape=[
            jax.ShapeDtypeStruct((NP, OUT_CH), jnp.float32),
            jax.ShapeDtypeStruct((NP, 1), jnp.float32),
            jax.ShapeDtypeStruct((NP, 1), jnp.float32),
        ],
    )(xp, W1, b1.reshape(1, HID_CH), W2, b2.reshape(1, OUT_CH), pd0, pd1)


def _upd_body(ps_r, v_r, v0_r, d2_r, o_r):
    s = ps_r[0] + ps_r[1] + v_r[...]
    o_r[...] = d2_r[...] * s + ALPHA * v0_r[...]


def _upd_last_body(ps_r, v_r, v0_r, d2_r, sq_r, o_r):
    s = ps_r[0] + ps_r[1] + v_r[...]
    o_r[...] = (d2_r[...] * s + ALPHA * v0_r[...]) * sq_r[...]


def _upd_call(ps, v, v0, d2, sq, last):
    in_specs = [
        pl.BlockSpec((NC, _BLK, OUT_CH), lambda i: (0, i, 0)),
        pl.BlockSpec((_BLK, OUT_CH), lambda i: (i, 0)),
        pl.BlockSpec((_BLK, OUT_CH), lambda i: (i, 0)),
        pl.BlockSpec((_BLK, 1), lambda i: (i, 0)),
    ]
    args = [ps, v, v0, d2]
    body = _upd_body
    if last:
        in_specs.append(pl.BlockSpec((_BLK, 1), lambda i: (i, 0)))
        args.append(sq)
        body = _upd_last_body
    return pl.pallas_call(
        body,
        grid=(16,),
        in_specs=in_specs,
        out_specs=pl.BlockSpec((_BLK, OUT_CH), lambda i: (i, 0)),
        out_shape=jax.ShapeDtypeStruct((NP, OUT_CH), jnp.float32),
    )(*args)


def kernel(x, edge_index, W1, b1, W2, b2):
    src = edge_index[0].reshape(NW, NCHUNK, CHUNK)
    dst = edge_index[1].reshape(NW, NCHUNK, CHUNK)

    pdeg = _deg_kernel(dst)
    pd0 = pdeg[0].reshape(NP, 1)
    pd1 = pdeg[1].reshape(NP, 1)

    xp = jnp.pad(x, ((0, NP - N), (0, 0)))
    v, d2, sq = _mlp_call(xp, W1, b1, W2, b2, pd0, pd1)
    v0 = v

    for t in range(K_PROP):
        ps = _prop_kernel(v, src, dst)
        v = _upd_call(ps, v, v0, d2, sq, last=(t == K_PROP - 1))

    return v[:N]


# speedup vs baseline: 18.1903x; 18.1903x over previous
import functools

import jax
import jax.numpy as jnp
from jax import lax
from jax.experimental import pallas as pl
from jax.experimental.pallas import tpu as pltpu
from jax.experimental.pallas import tpu_sc as plsc

N = 10000
E = 320000
IN_CH = 128
HID_CH = 128
OUT_CH = 64
K_PROP = 10
ALPHA = 0.1

NC = 2
NS = 16
NW = NC * NS
NP = 10240
RPT = NP // NS
EPT = E // NW
CHUNK = 80
NCHUNK = EPT // CHUNK

_MESH = plsc.VectorSubcoreMesh(core_axis_name="c", subcore_axis_name="s")


def _zero_vmem(ref, rows, width):
    z = jnp.zeros((16,), jnp.float32)

    def body(i, _):
        for k in range(width // 16):
            ref[i, pl.ds(k * 16, 16)] = z
        return 0

    lax.fori_loop(0, rows, body, 0)


def _deg_body(dst_hbm, pdeg_hbm, idx_v, ones_v, row_v, deg_sh):
    cid = lax.axis_index("c")
    sid = lax.axis_index("s")
    wid = cid * NS + sid

    pltpu.sync_copy(dst_hbm.at[wid], idx_v)
    one = jnp.ones((16,), jnp.float32)
    z = jnp.zeros((16,), jnp.float32)
    for k in range(CHUNK // 16):
        ones_v[pl.ds(k * 16, 16)] = one

    def zrow(i, _):
        row_v[pl.ds(i * 16, 16)] = z
        return 0

    lax.fori_loop(0, RPT // 16, zrow, 0)
    pltpu.sync_copy(row_v, deg_sh.at[pl.ds(sid * RPT, RPT)])
    plsc.subcore_barrier()

    def chunk(j, _):
        pltpu.sync_copy(ones_v, deg_sh.at[idx_v.at[j]], add=True)
        return 0

    lax.fori_loop(0, NCHUNK, chunk, 0)
    plsc.subcore_barrier()
    pltpu.sync_copy(deg_sh.at[pl.ds(sid * RPT, RPT)], row_v)
    pltpu.sync_copy(row_v, pdeg_hbm.at[cid, pl.ds(sid * RPT, RPT)])


_deg_kernel = functools.partial(
    pl.kernel,
    out_type=jax.ShapeDtypeStruct((NC, NP), jnp.float32),
    mesh=_MESH,
    compiler_params=pltpu.CompilerParams(use_tc_tiling_on_sc=False),
    scratch_types=[
        pltpu.VMEM((NCHUNK, CHUNK), jnp.int32),
        pltpu.VMEM((CHUNK,), jnp.float32),
        pltpu.VMEM((RPT,), jnp.float32),
        pltpu.VMEM_SHARED((NP,), jnp.float32),
    ],
)(_deg_body)


def _prop_body(v_hbm, src_hbm, dst_hbm, ps_hbm, idx_s, idx_d, gbuf0, gbuf1,
               zbuf, s_sh, sem0, sem1):
    cid = lax.axis_index("c")
    sid = lax.axis_index("s")
    wid = cid * NS + sid

    pltpu.sync_copy(src_hbm.at[wid], idx_s)
    pltpu.sync_copy(dst_hbm.at[wid], idx_d)

    _zero_vmem(zbuf, RPT // 4, OUT_CH)
    for q in range(4):
        pltpu.sync_copy(zbuf, s_sh.at[pl.ds(sid * RPT + q * (RPT // 4),
                                            RPT // 4)])
    plsc.subcore_barrier()

    pltpu.async_copy(v_hbm.at[idx_s.at[0]], gbuf0, sem0).wait()
    pltpu.async_copy(v_hbm.at[idx_s.at[1]], gbuf1, sem1)

    def chunk(i, _):
        j = 2 * i
        pltpu.sync_copy(gbuf0, s_sh.at[idx_d.at[j]], add=True)
        pltpu.make_async_copy(v_hbm.at[idx_s.at[j + 1]], gbuf1, sem1).wait()
        cp0 = pltpu.async_copy(v_hbm.at[idx_s.at[j + 2]], gbuf0, sem0)
        pltpu.sync_copy(gbuf1, s_sh.at[idx_d.at[j + 1]], add=True)
        cp0.wait()
        pltpu.async_copy(v_hbm.at[idx_s.at[j + 3]], gbuf1, sem1)
        return 0

    lax.fori_loop(0, (NCHUNK - 3) // 2, chunk, 0)
    pltpu.sync_copy(gbuf0, s_sh.at[idx_d.at[NCHUNK - 3]], add=True)
    cp = pltpu.async_copy(v_hbm.at[idx_s.at[NCHUNK - 1]], gbuf0, sem0)
    pltpu.make_async_copy(v_hbm.at[idx_s.at[NCHUNK - 2]], gbuf1, sem1).wait()
    pltpu.sync_copy(gbuf1, s_sh.at[idx_d.at[NCHUNK - 2]], add=True)
    cp.wait()
    pltpu.sync_copy(gbuf0, s_sh.at[idx_d.at[NCHUNK - 1]], add=True)

    plsc.subcore_barrier()
    for q in range(4):
        r0 = sid * RPT + q * (RPT // 4)
        pltpu.sync_copy(s_sh.at[pl.ds(r0, RPT // 4)], zbuf)
        pltpu.sync_copy(zbuf, ps_hbm.at[cid, pl.ds(r0, RPT // 4)])


_prop_kernel = functools.partial(
    pl.kernel,
    out_type=jax.ShapeDtypeStruct((NC, NP, OUT_CH), jnp.float32),
    mesh=_MESH,
    compiler_params=pltpu.CompilerParams(use_tc_tiling_on_sc=False),
    scratch_types=[
        pltpu.VMEM((NCHUNK, CHUNK), jnp.int32),
        pltpu.VMEM((NCHUNK, CHUNK), jnp.int32),
        pltpu.VMEM((CHUNK, OUT_CH), jnp.float32),
        pltpu.VMEM((CHUNK, OUT_CH), jnp.float32),
        pltpu.VMEM((RPT // 4, OUT_CH), jnp.float32),
        pltpu.VMEM_SHARED((NP, OUT_CH), jnp.float32),
        pltpu.SemaphoreType.DMA,
        pltpu.SemaphoreType.DMA,
    ],
)(_prop_body)


_BLK = NP // 16


def _mlp_body(x_r, w1_r, b1_r, w2_r, b2_r, pd0_r, pd1_r, v0_r, d2_r, sq_r):
    h = lax.dot_general(x_r[...], w1_r[...], (((1,), (1,)), ((), ())),
                        preferred_element_type=jnp.float32)
    h = jnp.maximum(h + b1_r[...], 0.0)
    h0 = lax.dot_general(h, w2_r[...], (((1,), (1,)), ((), ())),
                         preferred_element_type=jnp.float32) + b2_r[...]
    deg = pd0_r[...] + pd1_r[...] + 1.0
    dinv = lax.rsqrt(deg)
    d2_r[...] = 0.9 * dinv * dinv
    sq_r[...] = deg * dinv
    v0_r[...] = h0 * dinv


def _mlp_call(xp, W1, b1, W2, b2, pd0, pd1):
    return pl.pallas_call(
        _mlp_body,
        grid=(16,),
        in_specs=[
            pl.BlockSpec((_BLK, IN_CH), lambda i: (i, 0)),
            pl.BlockSpec((HID_CH, IN_CH), lambda i: (0, 0)),
            pl.BlockSpec((1, HID_CH), lambda i: (0, 0)),
            pl.BlockSpec((OUT_CH, HID_CH), lambda i: (0, 0)),
            pl.BlockSpec((1, OUT_CH), lambda i: (0, 0)),
            pl.BlockSpec((_BLK, 1), lambda i: (i, 0)),
            pl.BlockSpec((_BLK, 1), lambda i: (i, 0)),
        ],
        out_specs=[
            pl.BlockSpec((_BLK, OUT_CH), lambda i: (i, 0)),
            pl.BlockSpec((_BLK, 1), lambda i: (i, 0)),
            pl.BlockSpec((_BLK, 1), lambda i: (i, 0)),
        ],
        out_shape=[
            jax.ShapeDtypeStruct((NP, OUT_CH), jnp.float32),
            jax.ShapeDtypeStruct((NP, 1), jnp.float32),
            jax.ShapeDtypeStruct((NP, 1), jnp.float32),
        ],
    )(xp, W1, b1.reshape(1, HID_CH), W2, b2.reshape(1, OUT_CH), pd0, pd1)


def _upd_body(ps_r, v_r, v0_r, d2_r, o_r):
    s = ps_r[0] + ps_r[1] + v_r[...]
    o_r[...] = d2_r[...] * s + ALPHA * v0_r[...]


def _upd_last_body(ps_r, v_r, v0_r, d2_r, sq_r, o_r):
    s = ps_r[0] + ps_r[1] + v_r[...]
    o_r[...] = (d2_r[...] * s + ALPHA * v0_r[...]) * sq_r[...]


def _upd_call(ps, v, v0, d2, sq, last):
    in_specs = [
        pl.BlockSpec((NC, _BLK, OUT_CH), lambda i: (0, i, 0)),
        pl.BlockSpec((_BLK, OUT_CH), lambda i: (i, 0)),
        pl.BlockSpec((_BLK, OUT_CH), lambda i: (i, 0)),
        pl.BlockSpec((_BLK, 1), lambda i: (i, 0)),
    ]
    args = [ps, v, v0, d2]
    body = _upd_body
    if last:
        in_specs.append(pl.BlockSpec((_BLK, 1), lambda i: (i, 0)))
        args.append(sq)
        body = _upd_last_body
    return pl.pallas_call(
        body,
        grid=(16,),
        in_specs=in_specs,
        out_specs=pl.BlockSpec((_BLK, OUT_CH), lambda i: (i, 0)),
        out_shape=jax.ShapeDtypeStruct((NP, OUT_CH), jnp.float32),
    )(*args)


def kernel(x, edge_index, W1, b1, W2, b2):
    src = edge_index[0].reshape(NW, NCHUNK, CHUNK)
    dst = edge_index[1].reshape(NW, NCHUNK, CHUNK)

    pdeg = _deg_kernel(dst)
    pd0 = pdeg[0].reshape(NP, 1)
    pd1 = pdeg[1].reshape(NP, 1)

    xp = jnp.pad(x, ((0, NP - N), (0, 0)))
    v, d2, sq = _mlp_call(xp, W1, b1, W2, b2, pd0, pd1)
    v0 = v

    for t in range(K_PROP):
        ps = _prop_kernel(v, src, dst)
        v = _upd_call(ps, v, v0, d2, sq, last=(t == K_PROP - 1))

    return v[:N]


# speedup vs baseline: 28.3194x; 1.5568x over previous
import functools

import jax
import jax.numpy as jnp
from jax import lax
from jax.experimental import pallas as pl
from jax.experimental.pallas import tpu as pltpu
from jax.experimental.pallas import tpu_sc as plsc

N = 10000
E = 320000
IN_CH = 128
HID_CH = 128
OUT_CH = 64
K_PROP = 10
ALPHA = 0.1

NC = 2
NS = 16
NW = NC * NS
NP = 10240
RPT = NP // NS
CHUNK = 128
NCHUNK = 79
EPT = NCHUNK * CHUNK
E_PAD = NW * EPT
NBUF = 4

_MESH = plsc.VectorSubcoreMesh(core_axis_name="c", subcore_axis_name="s")


def _zero_vmem(ref, rows, width):
    z = jnp.zeros((16,), jnp.float32)

    def body(i, _):
        for k in range(width // 16):
            ref[i, pl.ds(k * 16, 16)] = z
        return 0

    lax.fori_loop(0, rows, body, 0)


def _deg_body(dst_hbm, pdeg_hbm, idx_v, ones_v, row_v, deg_sh):
    cid = lax.axis_index("c")
    sid = lax.axis_index("s")
    wid = cid * NS + sid

    pltpu.sync_copy(dst_hbm.at[wid], idx_v)
    one = jnp.ones((16,), jnp.float32)
    z = jnp.zeros((16,), jnp.float32)
    for k in range(CHUNK // 16):
        ones_v[pl.ds(k * 16, 16)] = one

    def zrow(i, _):
        row_v[pl.ds(i * 16, 16)] = z
        return 0

    lax.fori_loop(0, RPT // 16, zrow, 0)
    pltpu.sync_copy(row_v, deg_sh.at[pl.ds(sid * RPT, RPT)])
    plsc.subcore_barrier()

    def chunk(j, _):
        pltpu.sync_copy(ones_v, deg_sh.at[idx_v.at[j]], add=True)
        return 0

    lax.fori_loop(0, NCHUNK, chunk, 0)
    plsc.subcore_barrier()
    pltpu.sync_copy(deg_sh.at[pl.ds(sid * RPT, RPT)], row_v)
    pltpu.sync_copy(row_v, pdeg_hbm.at[cid, pl.ds(sid * RPT, RPT)])


_deg_kernel = functools.partial(
    pl.kernel,
    out_type=jax.ShapeDtypeStruct((NC, NP), jnp.float32),
    mesh=_MESH,
    compiler_params=pltpu.CompilerParams(use_tc_tiling_on_sc=False),
    scratch_types=[
        pltpu.VMEM((NCHUNK, CHUNK), jnp.int32),
        pltpu.VMEM((CHUNK,), jnp.float32),
        pltpu.VMEM((RPT,), jnp.float32),
        pltpu.VMEM_SHARED((NP,), jnp.float32),
    ],
)(_deg_body)


def _prop_body(v_hbm, src_hbm, dst_hbm, ps_hbm, idx_s, idx_d,
               gbuf0, gbuf1, gbuf2, gbuf3, zbuf, s_sh,
               gs0, gs1, gs2, gs3, ss0, ss1, ss2, ss3):
    cid = lax.axis_index("c")
    sid = lax.axis_index("s")
    wid = cid * NS + sid
    gbuf = (gbuf0, gbuf1, gbuf2, gbuf3)
    gsem = (gs0, gs1, gs2, gs3)
    ssem = (ss0, ss1, ss2, ss3)

    pltpu.sync_copy(src_hbm.at[wid], idx_s)
    pltpu.sync_copy(dst_hbm.at[wid], idx_d)

    _zero_vmem(zbuf, RPT // 4, OUT_CH)
    for q in range(4):
        pltpu.sync_copy(zbuf, s_sh.at[pl.ds(sid * RPT + q * (RPT // 4),
                                            RPT // 4)])
    plsc.subcore_barrier()

    def g_start(j, b):
        pltpu.async_copy(v_hbm.at[idx_s.at[j]], gbuf[b], gsem[b])

    def g_wait(j, b):
        pltpu.make_async_copy(v_hbm.at[idx_s.at[j]], gbuf[b], gsem[b]).wait()

    def s_start(j, b):
        pltpu.async_copy(gbuf[b], s_sh.at[idx_d.at[j]], ssem[b], add=True)

    def s_wait(j, b):
        pltpu.make_async_copy(gbuf[b], s_sh.at[idx_d.at[j]],
                              ssem[b]).wait()

    for b in range(NBUF):
        g_start(b, b)

    n_main = (NCHUNK - NBUF) // NBUF

    def superstep(i, _):
        j0 = i * NBUF
        for b in range(NBUF):
            g_wait(j0 + b, b)
            s_start(j0 + b, b)
        for b in range(NBUF):
            s_wait(j0 + b, b)
            g_start(j0 + NBUF + b, b)
        return 0

    lax.fori_loop(0, n_main, superstep, 0)
    for j in range(n_main * NBUF, NCHUNK):
        b = j % NBUF
        g_wait(j, b)
        s_start(j, b)
        if j + NBUF < NCHUNK:
            s_wait(j, b)
            g_start(j + NBUF, b)
    for j in range(NCHUNK - NBUF, NCHUNK):
        b = j % NBUF
        s_wait(j, b)

    plsc.subcore_barrier()
    for q in range(4):
        r0 = sid * RPT + q * (RPT // 4)
        pltpu.sync_copy(s_sh.at[pl.ds(r0, RPT // 4)], zbuf)
        pltpu.sync_copy(zbuf, ps_hbm.at[cid, pl.ds(r0, RPT // 4)])


_prop_kernel = functools.partial(
    pl.kernel,
    out_type=jax.ShapeDtypeStruct((NC, NP, OUT_CH), jnp.float32),
    mesh=_MESH,
    compiler_params=pltpu.CompilerParams(use_tc_tiling_on_sc=False),
    scratch_types=(
        [pltpu.VMEM((NCHUNK, CHUNK), jnp.int32)] * 2
        + [pltpu.VMEM((CHUNK, OUT_CH), jnp.float32)] * NBUF
        + [pltpu.VMEM((RPT // 4, OUT_CH), jnp.float32),
           pltpu.VMEM_SHARED((NP, OUT_CH), jnp.float32)]
        + [pltpu.SemaphoreType.DMA] * (2 * NBUF)
    ),
)(_prop_body)


_BLK = NP // 16


def _mlp_body(x_r, w1_r, b1_r, w2_r, b2_r, pd0_r, pd1_r, v0_r, d2_r, sq_r):
    h = lax.dot_general(x_r[...], w1_r[...], (((1,), (1,)), ((), ())),
                        preferred_element_type=jnp.float32)
    h = jnp.maximum(h + b1_r[...], 0.0)
    h0 = lax.dot_general(h, w2_r[...], (((1,), (1,)), ((), ())),
                         preferred_element_type=jnp.float32) + b2_r[...]
    deg = pd0_r[...] + pd1_r[...] + 1.0
    dinv = lax.rsqrt(deg)
    d2_r[...] = 0.9 * dinv * dinv
    sq_r[...] = deg * dinv
    v0_r[...] = h0 * dinv


def _mlp_call(xp, W1, b1, W2, b2, pd0, pd1):
    return pl.pallas_call(
        _mlp_body,
        grid=(16,),
        in_specs=[
            pl.BlockSpec((_BLK, IN_CH), lambda i: (i, 0)),
            pl.BlockSpec((HID_CH, IN_CH), lambda i: (0, 0)),
            pl.BlockSpec((1, HID_CH), lambda i: (0, 0)),
            pl.BlockSpec((OUT_CH, HID_CH), lambda i: (0, 0)),
            pl.BlockSpec((1, OUT_CH), lambda i: (0, 0)),
            pl.BlockSpec((_BLK, 1), lambda i: (i, 0)),
            pl.BlockSpec((_BLK, 1), lambda i: (i, 0)),
        ],
        out_specs=[
            pl.BlockSpec((_BLK, OUT_CH), lambda i: (i, 0)),
            pl.BlockSpec((_BLK, 1), lambda i: (i, 0)),
            pl.BlockSpec((_BLK, 1), lambda i: (i, 0)),
        ],
        out_shape=[
            jax.ShapeDtypeStruct((NP, OUT_CH), jnp.float32),
            jax.ShapeDtypeStruct((NP, 1), jnp.float32),
            jax.ShapeDtypeStruct((NP, 1), jnp.float32),
        ],
    )(xp, W1, b1.reshape(1, HID_CH), W2, b2.reshape(1, OUT_CH), pd0, pd1)


def _upd_body(ps_r, v_r, v0_r, d2_r, o_r):
    s = ps_r[0] + ps_r[1] + v_r[...]
    o_r[...] = d2_r[...] * s + ALPHA * v0_r[...]


def _upd_last_body(ps_r, v_r, v0_r, d2_r, sq_r, o_r):
    s = ps_r[0] + ps_r[1] + v_r[...]
    o_r[...] = (d2_r[...] * s + ALPHA * v0_r[...]) * sq_r[...]


def _upd_call(ps, v, v0, d2, sq, last):
    in_specs = [
        pl.BlockSpec((NC, _BLK, OUT_CH), lambda i: (0, i, 0)),
        pl.BlockSpec((_BLK, OUT_CH), lambda i: (i, 0)),
        pl.BlockSpec((_BLK, OUT_CH), lambda i: (i, 0)),
        pl.BlockSpec((_BLK, 1), lambda i: (i, 0)),
    ]
    args = [ps, v, v0, d2]
    body = _upd_body
    if last:
        in_specs.append(pl.BlockSpec((_BLK, 1), lambda i: (i, 0)))
        args.append(sq)
        body = _upd_last_body
    return pl.pallas_call(
        body,
        grid=(16,),
        in_specs=in_specs,
        out_specs=pl.BlockSpec((_BLK, OUT_CH), lambda i: (i, 0)),
        out_shape=jax.ShapeDtypeStruct((NP, OUT_CH), jnp.float32),
    )(*args)


def kernel(x, edge_index, W1, b1, W2, b2):
    pad = jnp.arange(E_PAD - E, dtype=jnp.int32)
    src = jnp.concatenate([edge_index[0], pad % N]).reshape(NW, NCHUNK, CHUNK)
    dst = jnp.concatenate([edge_index[1], N + pad % (NP - N)]).reshape(
        NW, NCHUNK, CHUNK)

    pdeg = _deg_kernel(dst)
    pd0 = pdeg[0].reshape(NP, 1)
    pd1 = pdeg[1].reshape(NP, 1)

    xp = jnp.pad(x, ((0, NP - N), (0, 0)))
    v, d2, sq = _mlp_call(xp, W1, b1, W2, b2, pd0, pd1)
    v0 = v

    for t in range(K_PROP):
        ps = _prop_kernel(v, src, dst)
        v = _upd_call(ps, v, v0, d2, sq, last=(t == K_PROP - 1))

    return v[:N]


# speedup vs baseline: 28.9554x; 1.0225x over previous
import functools

import jax
import jax.numpy as jnp
from jax import lax
from jax.experimental import pallas as pl
from jax.experimental.pallas import tpu as pltpu
from jax.experimental.pallas import tpu_sc as plsc

N = 10000
E = 320000
IN_CH = 128
HID_CH = 128
OUT_CH = 64
K_PROP = 10
ALPHA = 0.1

NC = 2
NS = 16
NW = NC * NS
NP = 10240
RPT = NP // NS
CHUNK = 128
NCHUNK = 79
EPT = NCHUNK * CHUNK
E_PAD = NW * EPT
NBUF = 6

_MESH = plsc.VectorSubcoreMesh(core_axis_name="c", subcore_axis_name="s")


def _zero_vmem(ref, rows, width):
    z = jnp.zeros((16,), jnp.float32)

    def body(i, _):
        for k in range(width // 16):
            ref[i, pl.ds(k * 16, 16)] = z
        return 0

    lax.fori_loop(0, rows, body, 0)


def _deg_body(dst_hbm, pdeg_hbm, idx_v, ones_v, row_v, deg_sh):
    cid = lax.axis_index("c")
    sid = lax.axis_index("s")
    wid = cid * NS + sid

    pltpu.sync_copy(dst_hbm.at[wid], idx_v)
    one = jnp.ones((16,), jnp.float32)
    z = jnp.zeros((16,), jnp.float32)
    for k in range(CHUNK // 16):
        ones_v[pl.ds(k * 16, 16)] = one

    def zrow(i, _):
        row_v[pl.ds(i * 16, 16)] = z
        return 0

    lax.fori_loop(0, RPT // 16, zrow, 0)
    pltpu.sync_copy(row_v, deg_sh.at[pl.ds(sid * RPT, RPT)])
    plsc.subcore_barrier()

    def chunk(j, _):
        pltpu.sync_copy(ones_v, deg_sh.at[idx_v.at[j]], add=True)
        return 0

    lax.fori_loop(0, NCHUNK, chunk, 0)
    plsc.subcore_barrier()
    pltpu.sync_copy(deg_sh.at[pl.ds(sid * RPT, RPT)], row_v)
    pltpu.sync_copy(row_v, pdeg_hbm.at[cid, pl.ds(sid * RPT, RPT)])


_deg_kernel = functools.partial(
    pl.kernel,
    out_type=jax.ShapeDtypeStruct((NC, NP), jnp.float32),
    mesh=_MESH,
    compiler_params=pltpu.CompilerParams(use_tc_tiling_on_sc=False),
    scratch_types=[
        pltpu.VMEM((NCHUNK, CHUNK), jnp.int32),
        pltpu.VMEM((CHUNK,), jnp.float32),
        pltpu.VMEM((RPT,), jnp.float32),
        pltpu.VMEM_SHARED((NP,), jnp.float32),
    ],
)(_deg_body)


def _prop_body(v_hbm, src_hbm, dst_hbm, ps_hbm, idx_s, idx_d, *rest):
    gbuf = rest[:NBUF]
    zbuf = rest[NBUF]
    s_sh = rest[NBUF + 1]
    gsem = rest[NBUF + 2:NBUF + 2 + NBUF]
    ssem = rest[NBUF + 2 + NBUF:]
    cid = lax.axis_index("c")
    sid = lax.axis_index("s")
    wid = cid * NS + sid

    pltpu.sync_copy(src_hbm.at[wid], idx_s)
    pltpu.sync_copy(dst_hbm.at[wid], idx_d)

    _zero_vmem(zbuf, RPT // 4, OUT_CH)
    for q in range(4):
        pltpu.sync_copy(zbuf, s_sh.at[pl.ds(sid * RPT + q * (RPT // 4),
                                            RPT // 4)])
    plsc.subcore_barrier()

    def g_start(j, b):
        pltpu.async_copy(v_hbm.at[idx_s.at[j]], gbuf[b], gsem[b])

    def g_wait(j, b):
        pltpu.make_async_copy(v_hbm.at[idx_s.at[j]], gbuf[b], gsem[b]).wait()

    def s_start(j, b):
        pltpu.async_copy(gbuf[b], s_sh.at[idx_d.at[j]], ssem[b], add=True)

    def s_wait(j, b):
        pltpu.make_async_copy(gbuf[b], s_sh.at[idx_d.at[j]],
                              ssem[b]).wait()

    for b in range(NBUF):
        g_start(b, b)

    n_main = (NCHUNK - NBUF) // NBUF

    def superstep(i, _):
        j0 = i * NBUF
        for b in range(NBUF):
            g_wait(j0 + b, b)
            s_start(j0 + b, b)
        for b in range(NBUF):
            s_wait(j0 + b, b)
            g_start(j0 + NBUF + b, b)
        return 0

    lax.fori_loop(0, n_main, superstep, 0)
    for j in range(n_main * NBUF, NCHUNK):
        b = j % NBUF
        g_wait(j, b)
        s_start(j, b)
        if j + NBUF < NCHUNK:
            s_wait(j, b)
            g_start(j + NBUF, b)
    for j in range(NCHUNK - NBUF, NCHUNK):
        b = j % NBUF
        s_wait(j, b)

    plsc.subcore_barrier()
    for q in range(4):
        r0 = sid * RPT + q * (RPT // 4)
        pltpu.sync_copy(s_sh.at[pl.ds(r0, RPT // 4)], zbuf)
        pltpu.sync_copy(zbuf, ps_hbm.at[cid, pl.ds(r0, RPT // 4)])


_prop_kernel = functools.partial(
    pl.kernel,
    out_type=jax.ShapeDtypeStruct((NC, NP, OUT_CH), jnp.float32),
    mesh=_MESH,
    compiler_params=pltpu.CompilerParams(use_tc_tiling_on_sc=False),
    scratch_types=(
        [pltpu.VMEM((NCHUNK, CHUNK), jnp.int32)] * 2
        + [pltpu.VMEM((CHUNK, OUT_CH), jnp.float32)] * NBUF
        + [pltpu.VMEM((RPT // 4, OUT_CH), jnp.float32),
           pltpu.VMEM_SHARED((NP, OUT_CH), jnp.float32)]
        + [pltpu.SemaphoreType.DMA] * (2 * NBUF)
    ),
)(_prop_body)


_BLK = NP // 16


def _mlp_body(x_r, w1_r, b1_r, w2_r, b2_r, pd0_r, pd1_r, v0_r, d2_r, sq_r):
    h = lax.dot_general(x_r[...], w1_r[...], (((1,), (1,)), ((), ())),
                        preferred_element_type=jnp.float32)
    h = jnp.maximum(h + b1_r[...], 0.0)
    h0 = lax.dot_general(h, w2_r[...], (((1,), (1,)), ((), ())),
                         preferred_element_type=jnp.float32) + b2_r[...]
    deg = pd0_r[...] + pd1_r[...] + 1.0
    dinv = lax.rsqrt(deg)
    d2_r[...] = 0.9 * dinv * dinv
    sq_r[...] = deg * dinv
    v0_r[...] = h0 * dinv


def _mlp_call(xp, W1, b1, W2, b2, pd0, pd1):
    return pl.pallas_call(
        _mlp_body,
        grid=(16,),
        in_specs=[
            pl.BlockSpec((_BLK, IN_CH), lambda i: (i, 0)),
            pl.BlockSpec((HID_CH, IN_CH), lambda i: (0, 0)),
            pl.BlockSpec((1, HID_CH), lambda i: (0, 0)),
            pl.BlockSpec((OUT_CH, HID_CH), lambda i: (0, 0)),
            pl.BlockSpec((1, OUT_CH), lambda i: (0, 0)),
            pl.BlockSpec((_BLK, 1), lambda i: (i, 0)),
            pl.BlockSpec((_BLK, 1), lambda i: (i, 0)),
        ],
        out_specs=[
            pl.BlockSpec((_BLK, OUT_CH), lambda i: (i, 0)),
            pl.BlockSpec((_BLK, 1), lambda i: (i, 0)),
            pl.BlockSpec((_BLK, 1), lambda i: (i, 0)),
        ],
        out_shape=[
            jax.ShapeDtypeStruct((NP, OUT_CH), jnp.float32),
            jax.ShapeDtypeStruct((NP, 1), jnp.float32),
            jax.ShapeDtypeStruct((NP, 1), jnp.float32),
        ],
    )(xp, W1, b1.reshape(1, HID_CH), W2, b2.reshape(1, OUT_CH), pd0, pd1)


def _upd_body(ps_r, v_r, v0_r, d2_r, o_r):
    s = ps_r[0] + ps_r[1] + v_r[...]
    o_r[...] = d2_r[...] * s + ALPHA * v0_r[...]


def _upd_last_body(ps_r, v_r, v0_r, d2_r, sq_r, o_r):
    s = ps_r[0] + ps_r[1] + v_r[...]
    o_r[...] = (d2_r[...] * s + ALPHA * v0_r[...]) * sq_r[...]


def _upd_call(ps, v, v0, d2, sq, last):
    in_specs = [
        pl.BlockSpec((NC, _BLK, OUT_CH), lambda i: (0, i, 0)),
        pl.BlockSpec((_BLK, OUT_CH), lambda i: (i, 0)),
        pl.BlockSpec((_BLK, OUT_CH), lambda i: (i, 0)),
        pl.BlockSpec((_BLK, 1), lambda i: (i, 0)),
    ]
    args = [ps, v, v0, d2]
    body = _upd_body
    if last:
        in_specs.append(pl.BlockSpec((_BLK, 1), lambda i: (i, 0)))
        args.append(sq)
        body = _upd_last_body
    return pl.pallas_call(
        body,
        grid=(16,),
        in_specs=in_specs,
        out_specs=pl.BlockSpec((_BLK, OUT_CH), lambda i: (i, 0)),
        out_shape=jax.ShapeDtypeStruct((NP, OUT_CH), jnp.float32),
    )(*args)


def kernel(x, edge_index, W1, b1, W2, b2):
    pad = jnp.arange(E_PAD - E, dtype=jnp.int32)
    src = jnp.concatenate([edge_index[0], pad % N]).reshape(NW, NCHUNK, CHUNK)
    dst = jnp.concatenate([edge_index[1], N + pad % (NP - N)]).reshape(
        NW, NCHUNK, CHUNK)

    pdeg = _deg_kernel(dst)
    pd0 = pdeg[0].reshape(NP, 1)
    pd1 = pdeg[1].reshape(NP, 1)

    xp = jnp.pad(x, ((0, NP - N), (0, 0)))
    v, d2, sq = _mlp_call(xp, W1, b1, W2, b2, pd0, pd1)
    v0 = v

    for t in range(K_PROP):
        ps = _prop_kernel(v, src, dst)
        v = _upd_call(ps, v, v0, d2, sq, last=(t == K_PROP - 1))

    return v[:N]


# speedup vs baseline: 30.1855x; 1.0425x over previous
import functools

import jax
import jax.numpy as jnp
from jax import lax
from jax.experimental import pallas as pl
from jax.experimental.pallas import tpu as pltpu
from jax.experimental.pallas import tpu_sc as plsc

N = 10000
E = 320000
IN_CH = 128
HID_CH = 128
OUT_CH = 64
K_PROP = 10
ALPHA = 0.1

NC = 2
NS = 16
NW = NC * NS
NP = 10240
RPT = NP // NS
CHUNK = 128
NCHUNK = 79
EPT = NCHUNK * CHUNK
E_PAD = NW * EPT
NBUF = 6

_MESH = plsc.VectorSubcoreMesh(core_axis_name="c", subcore_axis_name="s")


def _zero_vmem(ref, rows, width):
    z = jnp.zeros((16,), jnp.float32)

    def body(i, _):
        for k in range(width // 16):
            ref[i, pl.ds(k * 16, 16)] = z
        return 0

    lax.fori_loop(0, rows, body, 0)


def _deg_body(dst_hbm, pdeg_hbm, idx_v, ones_v, row_v, deg_sh):
    cid = lax.axis_index("c")
    sid = lax.axis_index("s")
    wid = cid * NS + sid

    pltpu.sync_copy(dst_hbm.at[wid], idx_v)
    one = jnp.ones((16,), jnp.float32)
    z = jnp.zeros((16,), jnp.float32)
    for k in range(CHUNK // 16):
        ones_v[pl.ds(k * 16, 16)] = one

    def zrow(i, _):
        row_v[pl.ds(i * 16, 16)] = z
        return 0

    lax.fori_loop(0, RPT // 16, zrow, 0)
    pltpu.sync_copy(row_v, deg_sh.at[pl.ds(sid * RPT, RPT)])
    plsc.subcore_barrier()

    def chunk(j, _):
        pltpu.sync_copy(ones_v, deg_sh.at[idx_v.at[j]], add=True)
        return 0

    lax.fori_loop(0, NCHUNK, chunk, 0)
    plsc.subcore_barrier()
    pltpu.sync_copy(deg_sh.at[pl.ds(sid * RPT, RPT)], row_v)
    pltpu.sync_copy(row_v, pdeg_hbm.at[cid, pl.ds(sid * RPT, RPT)])


_deg_kernel = functools.partial(
    pl.kernel,
    out_type=jax.ShapeDtypeStruct((NC, NP), jnp.float32),
    mesh=_MESH,
    compiler_params=pltpu.CompilerParams(use_tc_tiling_on_sc=False),
    scratch_types=[
        pltpu.VMEM((NCHUNK, CHUNK), jnp.int32),
        pltpu.VMEM((CHUNK,), jnp.float32),
        pltpu.VMEM((RPT,), jnp.float32),
        pltpu.VMEM_SHARED((NP,), jnp.float32),
    ],
)(_deg_body)


def _edge_ring(v_hbm, idx_s, idx_d, gbuf, gsem, ssem, s_sh, nbuf):
    def g_start(j, b):
        pltpu.async_copy(v_hbm.at[idx_s.at[j]], gbuf[b], gsem[b])

    def g_wait(j, b):
        pltpu.make_async_copy(v_hbm.at[idx_s.at[j]], gbuf[b], gsem[b]).wait()

    def s_start(j, b):
        pltpu.async_copy(gbuf[b], s_sh.at[idx_d.at[j]], ssem[b], add=True)

    def s_wait(j, b):
        pltpu.make_async_copy(gbuf[b], s_sh.at[idx_d.at[j]],
                              ssem[b]).wait()

    for b in range(nbuf):
        g_start(b, b)

    n_main = (NCHUNK - nbuf) // nbuf

    def superstep(i, _):
        j0 = i * nbuf
        for b in range(nbuf):
            g_wait(j0 + b, b)
            s_start(j0 + b, b)
        for b in range(nbuf):
            s_wait(j0 + b, b)
            g_start(j0 + nbuf + b, b)
        return 0

    lax.fori_loop(0, n_main, superstep, 0)
    for j in range(n_main * nbuf, NCHUNK):
        b = j % nbuf
        g_wait(j, b)
        s_start(j, b)
        if j + nbuf < NCHUNK:
            s_wait(j, b)
            g_start(j + nbuf, b)
    for j in range(NCHUNK - nbuf, NCHUNK):
        s_wait(j, j % nbuf)


def _ps_writeout(ps_hbm, s_sh, zbuf, cid, sid):
    for q in range(4):
        r0 = sid * RPT + q * (RPT // 4)
        pltpu.sync_copy(s_sh.at[pl.ds(r0, RPT // 4)], zbuf)
        pltpu.sync_copy(zbuf, ps_hbm.at[cid, pl.ds(r0, RPT // 4)])


def _zero_s_sh(s_sh, zbuf, sid):
    _zero_vmem(zbuf, RPT // 4, OUT_CH)
    for q in range(4):
        pltpu.sync_copy(zbuf, s_sh.at[pl.ds(sid * RPT + q * (RPT // 4),
                                            RPT // 4)])


def _prop_body(v_hbm, src_hbm, dst_hbm, ps_hbm, idx_s, idx_d, *rest):
    gbuf = rest[:NBUF]
    zbuf = rest[NBUF]
    s_sh = rest[NBUF + 1]
    gsem = rest[NBUF + 2:NBUF + 2 + NBUF]
    ssem = rest[NBUF + 2 + NBUF:]
    cid = lax.axis_index("c")
    sid = lax.axis_index("s")
    wid = cid * NS + sid

    pltpu.sync_copy(src_hbm.at[wid], idx_s)
    pltpu.sync_copy(dst_hbm.at[wid], idx_d)
    _zero_s_sh(s_sh, zbuf, sid)
    plsc.subcore_barrier()
    _edge_ring(v_hbm, idx_s, idx_d, gbuf, gsem, ssem, s_sh, NBUF)
    plsc.subcore_barrier()
    _ps_writeout(ps_hbm, s_sh, zbuf, cid, sid)


_prop_kernel = functools.partial(
    pl.kernel,
    out_type=jax.ShapeDtypeStruct((NC, NP, OUT_CH), jnp.float32),
    mesh=_MESH,
    compiler_params=pltpu.CompilerParams(use_tc_tiling_on_sc=False),
    scratch_types=(
        [pltpu.VMEM((NCHUNK, CHUNK), jnp.int32)] * 2
        + [pltpu.VMEM((CHUNK, OUT_CH), jnp.float32)] * NBUF
        + [pltpu.VMEM((RPT // 4, OUT_CH), jnp.float32),
           pltpu.VMEM_SHARED((NP, OUT_CH), jnp.float32)]
        + [pltpu.SemaphoreType.DMA] * (2 * NBUF)
    ),
)(_prop_body)


NBUF2 = 4
RCH = 128


def _prop_upd_body(psp_hbm, vp_hbm, v0_hbm, d2_hbm, src_hbm, dst_hbm,
                   ps_hbm, vout_hbm, idx_s, idx_d, *rest):
    gbuf = rest[:NBUF2]
    zbuf = rest[NBUF2]
    s_sh = rest[NBUF2 + 1]
    gsem = rest[NBUF2 + 2:NBUF2 + 2 + NBUF2]
    ssem = rest[NBUF2 + 2 + NBUF2:]
    cid = lax.axis_index("c")
    sid = lax.axis_index("s")
    wid = cid * NS + sid

    pltpu.sync_copy(src_hbm.at[wid], idx_s)
    pltpu.sync_copy(dst_hbm.at[wid], idx_d)
    _zero_s_sh(s_sh, zbuf, sid)

    g0, g1, g2, g3 = gbuf
    for q in range(RPT // RCH):
        r0 = sid * RPT + q * RCH
        rows = pl.ds(r0, RCH)
        pltpu.async_copy(psp_hbm.at[0, rows], g0, gsem[0])
        pltpu.async_copy(psp_hbm.at[1, rows], g1, gsem[1])
        pltpu.async_copy(vp_hbm.at[rows], g2, gsem[2])
        pltpu.async_copy(v0_hbm.at[rows], g3, gsem[3])
        pltpu.async_copy(d2_hbm.at[rows], zbuf.at[pl.ds(0, RCH)], ssem[0])
        pltpu.make_async_copy(psp_hbm.at[0, rows], g0, gsem[0]).wait()
        pltpu.make_async_copy(psp_hbm.at[1, rows], g1, gsem[1]).wait()
        pltpu.make_async_copy(vp_hbm.at[rows], g2, gsem[2]).wait()
        pltpu.make_async_copy(v0_hbm.at[rows], g3, gsem[3]).wait()
        pltpu.make_async_copy(d2_hbm.at[rows], zbuf.at[pl.ds(0, RCH)],
                              ssem[0]).wait()

        def prow(r, _):
            for k in range(OUT_CH // 16):
                sl = pl.ds(k * 16, 16)
                a = g0[r, sl] + g1[r, sl] + g2[r, sl]
                g0[r, sl] = zbuf[r, sl] * a + ALPHA * g3[r, sl]
            return 0

        lax.fori_loop(0, RCH, prow, 0)
        pltpu.sync_copy(g0, vout_hbm.at[rows])

    plsc.subcore_barrier()
    _edge_ring(vout_hbm, idx_s, idx_d, gbuf, gsem, ssem, s_sh, NBUF2)
    plsc.subcore_barrier()
    _ps_writeout(ps_hbm, s_sh, zbuf, cid, sid)


_prop_upd_kernel = functools.partial(
    pl.kernel,
    out_type=(jax.ShapeDtypeStruct((NC, NP, OUT_CH), jnp.float32),
              jax.ShapeDtypeStruct((NP, OUT_CH), jnp.float32)),
    mesh=_MESH,
    compiler_params=pltpu.CompilerParams(use_tc_tiling_on_sc=False),
    scratch_types=(
        [pltpu.VMEM((NCHUNK, CHUNK), jnp.int32)] * 2
        + [pltpu.VMEM((CHUNK, OUT_CH), jnp.float32)] * NBUF2
        + [pltpu.VMEM((RPT // 4, OUT_CH), jnp.float32),
           pltpu.VMEM_SHARED((NP, OUT_CH), jnp.float32)]
        + [pltpu.SemaphoreType.DMA] * (2 * NBUF2)
    ),
)(_prop_upd_body)


_BLK = NP // 16


def _mlp_body(x_r, w1_r, b1_r, w2_r, b2_r, pd0_r, pd1_r, v0_r, d2_r, sq_r):
    h = lax.dot_general(x_r[...], w1_r[...], (((1,), (1,)), ((), ())),
                        preferred_element_type=jnp.float32)
    h = jnp.maximum(h + b1_r[...], 0.0)
    h0 = lax.dot_general(h, w2_r[...], (((1,), (1,)), ((), ())),
                         preferred_element_type=jnp.float32) + b2_r[...]
    deg = pd0_r[...] + pd1_r[...] + 1.0
    dinv = lax.rsqrt(deg)
    d2_r[...] = jnp.broadcast_to(0.9 * dinv * dinv, (_BLK, OUT_CH))
    sq_r[...] = deg * dinv
    v0_r[...] = h0 * dinv


def _mlp_call(xp, W1, b1, W2, b2, pd0, pd1):
    return pl.pallas_call(
        _mlp_body,
        grid=(16,),
        in_specs=[
            pl.BlockSpec((_BLK, IN_CH), lambda i: (i, 0)),
            pl.BlockSpec((HID_CH, IN_CH), lambda i: (0, 0)),
            pl.BlockSpec((1, HID_CH), lambda i: (0, 0)),
            pl.BlockSpec((OUT_CH, HID_CH), lambda i: (0, 0)),
            pl.BlockSpec((1, OUT_CH), lambda i: (0, 0)),
            pl.BlockSpec((_BLK, 1), lambda i: (i, 0)),
            pl.BlockSpec((_BLK, 1), lambda i: (i, 0)),
        ],
        out_specs=[
            pl.BlockSpec((_BLK, OUT_CH), lambda i: (i, 0)),
            pl.BlockSpec((_BLK, OUT_CH), lambda i: (i, 0)),
            pl.BlockSpec((_BLK, 1), lambda i: (i, 0)),
        ],
        out_shape=[
            jax.ShapeDtypeStruct((NP, OUT_CH), jnp.float32),
            jax.ShapeDtypeStruct((NP, OUT_CH), jnp.float32),
            jax.ShapeDtypeStruct((NP, 1), jnp.float32),
        ],
    )(xp, W1, b1.reshape(1, HID_CH), W2, b2.reshape(1, OUT_CH), pd0, pd1)


def _upd_last_body(ps_r, v_r, v0_r, d2_r, sq_r, o_r):
    s = ps_r[0] + ps_r[1] + v_r[...]
    o_r[...] = (d2_r[...] * s + ALPHA * v0_r[...]) * sq_r[...]


def _upd_call(ps, v, v0, d2, sq):
    return pl.pallas_call(
        _upd_last_body,
        grid=(16,),
        in_specs=[
            pl.BlockSpec((NC, _BLK, OUT_CH), lambda i: (0, i, 0)),
            pl.BlockSpec((_BLK, OUT_CH), lambda i: (i, 0)),
            pl.BlockSpec((_BLK, OUT_CH), lambda i: (i, 0)),
            pl.BlockSpec((_BLK, OUT_CH), lambda i: (i, 0)),
            pl.BlockSpec((_BLK, 1), lambda i: (i, 0)),
        ],
        out_specs=pl.BlockSpec((_BLK, OUT_CH), lambda i: (i, 0)),
        out_shape=jax.ShapeDtypeStruct((NP, OUT_CH), jnp.float32),
    )(ps, v, v0, d2, sq)


def kernel(x, edge_index, W1, b1, W2, b2):
    pad = jnp.arange(E_PAD - E, dtype=jnp.int32)
    src = jnp.concatenate([edge_index[0], pad % N]).reshape(NW, NCHUNK, CHUNK)
    dst = jnp.concatenate([edge_index[1], N + pad % (NP - N)]).reshape(
        NW, NCHUNK, CHUNK)

    pdeg = _deg_kernel(dst)
    pd0 = pdeg[0].reshape(NP, 1)
    pd1 = pdeg[1].reshape(NP, 1)

    xp = jnp.pad(x, ((0, NP - N), (0, 0)))
    v0, d2, sq = _mlp_call(xp, W1, b1, W2, b2, pd0, pd1)

    ps = _prop_kernel(v0, src, dst)
    v = v0
    for _ in range(K_PROP - 1):
        ps, v = _prop_upd_kernel(ps, v, v0, d2, src, dst)
    h = _upd_call(ps, v, v0, d2, sq)
    return h[:N]


# speedup vs baseline: 30.9724x; 1.0261x over previous
import functools

import jax
import jax.numpy as jnp
from jax import lax
from jax.experimental import pallas as pl
from jax.experimental.pallas import tpu as pltpu
from jax.experimental.pallas import tpu_sc as plsc

N = 10000
E = 320000
IN_CH = 128
HID_CH = 128
OUT_CH = 64
K_PROP = 10
ALPHA = 0.1

NC = 2
NS = 16
NW = NC * NS
NP = 10240
RPT = NP // NS
CHUNK = 128
NCHUNK = 79
EPT = NCHUNK * CHUNK
E_PAD = NW * EPT
NBUF = 6

_MESH = plsc.VectorSubcoreMesh(core_axis_name="c", subcore_axis_name="s")


def _zero_vmem(ref, rows, width):
    z = jnp.zeros((16,), jnp.float32)

    def body(i, _):
        for k in range(width // 16):
            ref[i, pl.ds(k * 16, 16)] = z
        return 0

    lax.fori_loop(0, rows, body, 0)


def _deg_body(dst_hbm, pdeg_hbm, idx_v, ones_v, row_v, deg_sh):
    cid = lax.axis_index("c")
    sid = lax.axis_index("s")
    wid = cid * NS + sid

    pltpu.sync_copy(dst_hbm.at[wid], idx_v)
    one = jnp.ones((16,), jnp.float32)
    z = jnp.zeros((16,), jnp.float32)
    for k in range(CHUNK // 16):
        ones_v[pl.ds(k * 16, 16)] = one

    def zrow(i, _):
        row_v[pl.ds(i * 16, 16)] = z
        return 0

    lax.fori_loop(0, RPT // 16, zrow, 0)
    pltpu.sync_copy(row_v, deg_sh.at[pl.ds(sid * RPT, RPT)])
    plsc.subcore_barrier()

    def chunk(j, _):
        pltpu.sync_copy(ones_v, deg_sh.at[idx_v.at[j]], add=True)
        return 0

    lax.fori_loop(0, NCHUNK, chunk, 0)
    plsc.subcore_barrier()
    pltpu.sync_copy(deg_sh.at[pl.ds(sid * RPT, RPT)], row_v)
    pltpu.sync_copy(row_v, pdeg_hbm.at[cid, pl.ds(sid * RPT, RPT)])


_deg_kernel = functools.partial(
    pl.kernel,
    out_type=jax.ShapeDtypeStruct((NC, NP), jnp.float32),
    mesh=_MESH,
    compiler_params=pltpu.CompilerParams(use_tc_tiling_on_sc=False),
    scratch_types=[
        pltpu.VMEM((NCHUNK, CHUNK), jnp.int32),
        pltpu.VMEM((CHUNK,), jnp.float32),
        pltpu.VMEM((RPT,), jnp.float32),
        pltpu.VMEM_SHARED((NP,), jnp.float32),
    ],
)(_deg_body)


def _edge_ring(v_hbm, idx_s, idx_d, gbuf, gsem, ssem, s_sh, nbuf):
    def g_start(j, b):
        pltpu.async_copy(v_hbm.at[idx_s.at[j]], gbuf[b], gsem[b])

    def g_wait(j, b):
        pltpu.make_async_copy(v_hbm.at[idx_s.at[j]], gbuf[b], gsem[b]).wait()

    def s_start(j, b):
        pltpu.async_copy(gbuf[b], s_sh.at[idx_d.at[j]], ssem[b], add=True)

    def s_wait(j, b):
        pltpu.make_async_copy(gbuf[b], s_sh.at[idx_d.at[j]],
                              ssem[b]).wait()

    for b in range(nbuf):
        g_start(b, b)

    n_main = (NCHUNK - nbuf) // nbuf

    def superstep(i, _):
        j0 = i * nbuf
        for b in range(nbuf):
            g_wait(j0 + b, b)
            s_start(j0 + b, b)
        for b in range(nbuf):
            s_wait(j0 + b, b)
            g_start(j0 + nbuf + b, b)
        return 0

    lax.fori_loop(0, n_main, superstep, 0)
    for j in range(n_main * nbuf, NCHUNK):
        b = j % nbuf
        g_wait(j, b)
        s_start(j, b)
        if j + nbuf < NCHUNK:
            s_wait(j, b)
            g_start(j + nbuf, b)
    for j in range(NCHUNK - nbuf, NCHUNK):
        s_wait(j, j % nbuf)


def _ps_writeout(ps_hbm, s_sh, zbuf, cid, sid):
    for q in range(4):
        r0 = sid * RPT + q * (RPT // 4)
        pltpu.sync_copy(s_sh.at[pl.ds(r0, RPT // 4)], zbuf)
        pltpu.sync_copy(zbuf, ps_hbm.at[cid, pl.ds(r0, RPT // 4)])


def _zero_s_sh(s_sh, zbuf, sid):
    _zero_vmem(zbuf, RPT // 4, OUT_CH)
    for q in range(4):
        pltpu.sync_copy(zbuf, s_sh.at[pl.ds(sid * RPT + q * (RPT // 4),
                                            RPT // 4)])


def _prop_body(v_hbm, src_hbm, dst_hbm, ps_hbm, idx_s, idx_d, *rest):
    gbuf = rest[:NBUF]
    zbuf = rest[NBUF]
    s_sh = rest[NBUF + 1]
    gsem = rest[NBUF + 2:NBUF + 2 + NBUF]
    ssem = rest[NBUF + 2 + NBUF:]
    cid = lax.axis_index("c")
    sid = lax.axis_index("s")
    wid = cid * NS + sid

    pltpu.sync_copy(src_hbm.at[wid], idx_s)
    pltpu.sync_copy(dst_hbm.at[wid], idx_d)
    _zero_s_sh(s_sh, zbuf, sid)
    plsc.subcore_barrier()
    _edge_ring(v_hbm, idx_s, idx_d, gbuf, gsem, ssem, s_sh, NBUF)
    plsc.subcore_barrier()
    _ps_writeout(ps_hbm, s_sh, zbuf, cid, sid)


_prop_kernel = functools.partial(
    pl.kernel,
    out_type=jax.ShapeDtypeStruct((NC, NP, OUT_CH), jnp.float32),
    mesh=_MESH,
    compiler_params=pltpu.CompilerParams(use_tc_tiling_on_sc=False),
    scratch_types=(
        [pltpu.VMEM((NCHUNK, CHUNK), jnp.int32)] * 2
        + [pltpu.VMEM((CHUNK, OUT_CH), jnp.float32)] * NBUF
        + [pltpu.VMEM((RPT // 4, OUT_CH), jnp.float32),
           pltpu.VMEM_SHARED((NP, OUT_CH), jnp.float32)]
        + [pltpu.SemaphoreType.DMA] * (2 * NBUF)
    ),
)(_prop_body)


NBUF2 = 6
RCH = 128


def _prop_upd_body(psp_hbm, vp_hbm, v0_hbm, d2_hbm, src_hbm, dst_hbm,
                   ps_hbm, vout_hbm, idx_s, idx_d, *rest):
    gbuf = rest[:NBUF2]
    zbuf = rest[NBUF2]
    s_sh = rest[NBUF2 + 1]
    gsem = rest[NBUF2 + 2:NBUF2 + 2 + NBUF2]
    ssem = rest[NBUF2 + 2 + NBUF2:]
    cid = lax.axis_index("c")
    sid = lax.axis_index("s")
    wid = cid * NS + sid

    pltpu.sync_copy(src_hbm.at[wid], idx_s)
    pltpu.sync_copy(dst_hbm.at[wid], idx_d)
    _zero_s_sh(s_sh, zbuf, sid)

    g0, g1, g2, g3 = gbuf[:4]
    for q in range(RPT // RCH):
        r0 = sid * RPT + q * RCH
        rows = pl.ds(r0, RCH)
        pltpu.async_copy(psp_hbm.at[0, rows], g0, gsem[0])
        pltpu.async_copy(psp_hbm.at[1, rows], g1, gsem[1])
        pltpu.async_copy(vp_hbm.at[rows], g2, gsem[2])
        pltpu.async_copy(v0_hbm.at[rows], g3, gsem[3])
        pltpu.async_copy(d2_hbm.at[rows], zbuf.at[pl.ds(0, RCH)], ssem[0])
        pltpu.make_async_copy(psp_hbm.at[0, rows], g0, gsem[0]).wait()
        pltpu.make_async_copy(psp_hbm.at[1, rows], g1, gsem[1]).wait()
        pltpu.make_async_copy(vp_hbm.at[rows], g2, gsem[2]).wait()
        pltpu.make_async_copy(v0_hbm.at[rows], g3, gsem[3]).wait()
        pltpu.make_async_copy(d2_hbm.at[rows], zbuf.at[pl.ds(0, RCH)],
                              ssem[0]).wait()

        def prow(r, _):
            for k in range(OUT_CH // 16):
                sl = pl.ds(k * 16, 16)
                a = g0[r, sl] + g1[r, sl] + g2[r, sl]
                g0[r, sl] = zbuf[r, sl] * a + ALPHA * g3[r, sl]
            return 0

        lax.fori_loop(0, RCH, prow, 0)
        pltpu.sync_copy(g0, vout_hbm.at[rows])

    plsc.subcore_barrier()
    _edge_ring(vout_hbm, idx_s, idx_d, gbuf, gsem, ssem, s_sh, NBUF2)
    plsc.subcore_barrier()
    _ps_writeout(ps_hbm, s_sh, zbuf, cid, sid)


_prop_upd_kernel = functools.partial(
    pl.kernel,
    out_type=(jax.ShapeDtypeStruct((NC, NP, OUT_CH), jnp.float32),
              jax.ShapeDtypeStruct((NP, OUT_CH), jnp.float32)),
    mesh=_MESH,
    compiler_params=pltpu.CompilerParams(use_tc_tiling_on_sc=False),
    scratch_types=(
        [pltpu.VMEM((NCHUNK, CHUNK), jnp.int32)] * 2
        + [pltpu.VMEM((CHUNK, OUT_CH), jnp.float32)] * NBUF2
        + [pltpu.VMEM((RPT // 4, OUT_CH), jnp.float32),
           pltpu.VMEM_SHARED((NP, OUT_CH), jnp.float32)]
        + [pltpu.SemaphoreType.DMA] * (2 * NBUF2)
    ),
)(_prop_upd_body)


_BLK = NP // 16


def _mlp_body(x_r, w1_r, b1_r, w2_r, b2_r, pd0_r, pd1_r, v0_r, d2_r, sq_r):
    h = lax.dot_general(x_r[...], w1_r[...], (((1,), (1,)), ((), ())),
                        preferred_element_type=jnp.float32)
    h = jnp.maximum(h + b1_r[...], 0.0)
    h0 = lax.dot_general(h, w2_r[...], (((1,), (1,)), ((), ())),
                         preferred_element_type=jnp.float32) + b2_r[...]
    deg = pd0_r[...] + pd1_r[...] + 1.0
    dinv = lax.rsqrt(deg)
    d2_r[...] = jnp.broadcast_to(0.9 * dinv * dinv, (_BLK, OUT_CH))
    sq_r[...] = deg * dinv
    v0_r[...] = h0 * dinv


def _mlp_call(xp, W1, b1, W2, b2, pd0, pd1):
    return pl.pallas_call(
        _mlp_body,
        grid=(16,),
        in_specs=[
            pl.BlockSpec((_BLK, IN_CH), lambda i: (i, 0)),
            pl.BlockSpec((HID_CH, IN_CH), lambda i: (0, 0)),
            pl.BlockSpec((1, HID_CH), lambda i: (0, 0)),
            pl.BlockSpec((OUT_CH, HID_CH), lambda i: (0, 0)),
            pl.BlockSpec((1, OUT_CH), lambda i: (0, 0)),
            pl.BlockSpec((_BLK, 1), lambda i: (i, 0)),
            pl.BlockSpec((_BLK, 1), lambda i: (i, 0)),
        ],
        out_specs=[
            pl.BlockSpec((_BLK, OUT_CH), lambda i: (i, 0)),
            pl.BlockSpec((_BLK, OUT_CH), lambda i: (i, 0)),
            pl.BlockSpec((_BLK, 1), lambda i: (i, 0)),
        ],
        out_shape=[
            jax.ShapeDtypeStruct((NP, OUT_CH), jnp.float32),
            jax.ShapeDtypeStruct((NP, OUT_CH), jnp.float32),
            jax.ShapeDtypeStruct((NP, 1), jnp.float32),
        ],
    )(xp, W1, b1.reshape(1, HID_CH), W2, b2.reshape(1, OUT_CH), pd0, pd1)


def _upd_last_body(ps_r, v_r, v0_r, d2_r, sq_r, o_r):
    s = ps_r[0] + ps_r[1] + v_r[...]
    o_r[...] = (d2_r[...] * s + ALPHA * v0_r[...]) * sq_r[...]


def _upd_call(ps, v, v0, d2, sq):
    return pl.pallas_call(
        _upd_last_body,
        grid=(16,),
        in_specs=[
            pl.BlockSpec((NC, _BLK, OUT_CH), lambda i: (0, i, 0)),
            pl.BlockSpec((_BLK, OUT_CH), lambda i: (i, 0)),
            pl.BlockSpec((_BLK, OUT_CH), lambda i: (i, 0)),
            pl.BlockSpec((_BLK, OUT_CH), lambda i: (i, 0)),
            pl.BlockSpec((_BLK, 1), lambda i: (i, 0)),
        ],
        out_specs=pl.BlockSpec((_BLK, OUT_CH), lambda i: (i, 0)),
        out_shape=jax.ShapeDtypeStruct((NP, OUT_CH), jnp.float32),
    )(ps, v, v0, d2, sq)


def kernel(x, edge_index, W1, b1, W2, b2):
    pad = jnp.arange(E_PAD - E, dtype=jnp.int32)
    src = jnp.concatenate([edge_index[0], pad % N]).reshape(NW, NCHUNK, CHUNK)
    dst = jnp.concatenate([edge_index[1], N + pad % (NP - N)]).reshape(
        NW, NCHUNK, CHUNK)

    pdeg = _deg_kernel(dst)
    pd0 = pdeg[0].reshape(NP, 1)
    pd1 = pdeg[1].reshape(NP, 1)

    xp = jnp.pad(x, ((0, NP - N), (0, 0)))
    v0, d2, sq = _mlp_call(xp, W1, b1, W2, b2, pd0, pd1)

    ps = _prop_kernel(v0, src, dst)
    v = v0
    for _ in range(K_PROP - 1):
        ps, v = _prop_upd_kernel(ps, v, v0, d2, src, dst)
    h = _upd_call(ps, v, v0, d2, sq)
    return h[:N]
